# baseline (speedup 1.0000x reference)
import functools

import jax
import jax.numpy as jnp
from jax import lax
from jax.experimental import pallas as pl
from jax.experimental.pallas import tpu as pltpu
from jax.experimental.pallas import tpu_sc as plsc

TOPK = 64


def _enc_body(x_ref, w_ref, b_ref, out_ref):
    acc = jnp.dot(x_ref[...], w_ref[...], preferred_element_type=jnp.float32)
    out_ref[...] = acc + b_ref[...]


def _encode(x, W_enc, b_enc):
    B, K = x.shape
    H = W_enc.shape[1]
    bh = 1024
    return pl.pallas_call(
        _enc_body,
        grid=(H // bh,),
        in_specs=[
            pl.BlockSpec((B, K), lambda j: (0, 0)),
            pl.BlockSpec((K, bh), lambda j: (0, j)),
            pl.BlockSpec((bh,), lambda j: (j,)),
        ],
        out_specs=pl.BlockSpec((B, bh), lambda j: (0, j)),
        out_shape=jax.ShapeDtypeStruct((B, H), jnp.float32),
    )(x, W_enc, b_enc)


def _sc_threshold_kernel(B, H):
    mesh = plsc.VectorSubcoreMesh(core_axis_name="c", subcore_axis_name="s")
    info = plsc.get_sparse_core_info()
    NC, NS, L = info.num_cores, info.num_subcores, info.num_lanes
    NW = NC * NS
    rows_per_w = B // NW
    nv_row = H // L

    @functools.partial(
        pl.kernel,
        mesh=mesh,
        out_type=jax.ShapeDtypeStruct((B,), jnp.float32),
        compiler_params=pltpu.CompilerParams(needs_layout_passes=False),
        scratch_types=[
            pltpu.VMEM((H,), jnp.float32),
            pltpu.VMEM((H + L,), jnp.int32),
            pltpu.VMEM((256,), jnp.int32),
            pltpu.VMEM((rows_per_w,), jnp.float32),
        ],
    )
    def body(pre_hbm, t_hbm, rowbuf, cand, hist, tbuf):
        wid = lax.axis_index("s") * NC + lax.axis_index("c")
        base = wid * rows_per_w
        zeros16i = jnp.zeros((L,), jnp.int32)
        ones16i = jnp.ones((L,), jnp.int32)
        iota16 = lax.iota(jnp.int32, L)

        def do_row(i, tvec):
            row = base + i
            pltpu.sync_copy(pre_hbm.at[row], rowbuf)

            def zh(u, c):
                hist[pl.ds(u * L, L)] = zeros16i
                return c

            lax.fori_loop(0, 256 // L, zh, 0, unroll=True)

            def p1(j, c):
                v = rowbuf[pl.ds(j * L, L)]
                h = jnp.maximum(v, 0.0)
                bits = lax.bitcast_convert_type(h, jnp.int32)
                e = lax.shift_right_logical(bits, 23)
                plsc.addupdate_scatter(hist, [e], ones16i)
                return c

            lax.fori_loop(0, nv_row, p1, 0, unroll=8)

            def scan(k, carry):
                S, estar, Cab = carry
                j = (256 // L) - 1 - k
                v = hist[pl.ds(j * L, L)]
                tot = jnp.sum(v)
                rev = lax.rev(v, dimensions=(0,))
                csum = plsc.cumsum(rev)
                crossed = (S + csum) >= TOPK
                ncross = plsc.all_reduce_population_count(crossed)[0]
                has = ncross > 0
                p = plsc.all_reduce_ffs(crossed)[0]
                bin_idx = j * L + (L - 1) - p
                cprev = jnp.sum(jnp.where(iota16 == p - 1, csum, 0))
                Cnew = S + jnp.where(p > 0, cprev, 0)
                found_now = jnp.logical_and(has, estar < 0)
                estar = jnp.where(found_now, bin_idx, estar)
                Cab = jnp.where(found_now, Cnew, Cab)
                return S + tot, estar, Cab

            _, estar, Cab = lax.fori_loop(
                0, 256 // L, scan, (jnp.int32(0), jnp.int32(-1), jnp.int32(0))
            )
            m = TOPK - Cab

            def p2(j, ptr):
                v = rowbuf[pl.ds(j * L, L)]
                h = jnp.maximum(v, 0.0)
                bits = lax.bitcast_convert_type(h, jnp.int32)
                e = lax.shift_right_logical(bits, 23)
                msk = e == estar
                plsc.store_compressed(cand.at[pl.ds(ptr, L)], bits, mask=msk)
                return ptr + plsc.all_reduce_population_count(msk)[0]

            ncand = lax.fori_loop(0, nv_row, p2, jnp.int32(0), unroll=8)
            cand[pl.ds(ncand, L)] = zeros16i
            nvc = (ncand + (L - 1)) // L

            lo0 = lax.shift_left(estar, 23)
            hi0 = lax.shift_left(estar + 1, 23)

            def bs(_, carry):
                lo, hi = carry
                mid = lo + lax.shift_right_logical(hi - lo, 1)

                def cnt_b(j, acc):
                    cv = cand[pl.ds(j * L, L)]
                    return acc + plsc.all_reduce_population_count(cv >= mid)[0]

                cnt = lax.fori_loop(0, nvc, cnt_b, jnp.int32(0))
                ge = cnt >= m
                return jnp.where(ge, mid, lo), jnp.where(ge, hi, mid)

            lo, _ = lax.fori_loop(0, 24, bs, (lo0, hi0))
            t = lax.bitcast_convert_type(jnp.broadcast_to(lo, (L,)), jnp.float32)
            return jnp.where(iota16 == (i % L), t, tvec)

        ngroups = rows_per_w // L

        def do_group(g, c):
            tvec = lax.fori_loop(
                g * L, (g + 1) * L, do_row, jnp.zeros((L,), jnp.float32)
            )
            tbuf[pl.ds(g * L, L)] = tvec
            return c

        lax.fori_loop(0, ngroups, do_group, 0)
        pltpu.sync_copy(tbuf, t_hbm.at[pl.ds(base, rows_per_w)])

    return body


def _sc_threshold(preact):
    B, H = preact.shape
    return _sc_threshold_kernel(B, H)(preact)


def _dec_body(pre_ref, t_ref, w_ref, b_ref, hs_ref, out_ref):
    k = pl.program_id(0)
    nk = pl.num_programs(0)
    h = jnp.maximum(pre_ref[...], 0.0)
    hs = jnp.where(h >= t_ref[...], h, 0.0)
    hs_ref[...] = hs
    acc = jnp.dot(hs, w_ref[...], preferred_element_type=jnp.float32)

    @pl.when(k == 0)
    def _():
        out_ref[...] = acc

    @pl.when(k != 0)
    def _():
        out_ref[...] = out_ref[...] + acc

    @pl.when(k == nk - 1)
    def _():
        out_ref[...] = jnp.maximum(out_ref[...] + b_ref[...], 0.0)


def _decode(preact, t, W_dec, b_dec):
    B, H = preact.shape
    D = W_dec.shape[1]
    bk = 1024
    return pl.pallas_call(
        _dec_body,
        grid=(H // bk,),
        in_specs=[
            pl.BlockSpec((B, bk), lambda k: (0, k)),
            pl.BlockSpec((B, 1), lambda k: (0, 0)),
            pl.BlockSpec((bk, D), lambda k: (k, 0)),
            pl.BlockSpec((D,), lambda k: (0,)),
        ],
        out_specs=[
            pl.BlockSpec((B, bk), lambda k: (0, k)),
            pl.BlockSpec((B, D), lambda k: (0, 0)),
        ],
        out_shape=[
            jax.ShapeDtypeStruct((B, H), jnp.float32),
            jax.ShapeDtypeStruct((B, D), jnp.float32),
        ],
    )(preact, t, W_dec, b_dec)


def kernel(x, W_enc, W_dec, b_enc, b_dec):
    preact = _encode(x, W_enc, b_enc)
    t = _sc_threshold(preact)
    hidden_sparse, out = _decode(preact, t.reshape(-1, 1), W_dec, b_dec)
    return (out, hidden_sparse, preact)


# speedup vs baseline: 1.3119x; 1.3119x over previous
import functools

import jax
import jax.numpy as jnp
from jax import lax
from jax.experimental import pallas as pl
from jax.experimental.pallas import tpu as pltpu
from jax.experimental.pallas import tpu_sc as plsc

TOPK = 64


def _enc_body(x_ref, w_ref, b_ref, out_ref):
    acc = jnp.dot(x_ref[...], w_ref[...], preferred_element_type=jnp.float32)
    out_ref[...] = acc + b_ref[...]


def _encode(x, W_enc, b_enc):
    B, K = x.shape
    H = W_enc.shape[1]
    bh = 1024
    return pl.pallas_call(
        _enc_body,
        grid=(H // bh,),
        in_specs=[
            pl.BlockSpec((B, K), lambda j: (0, 0)),
            pl.BlockSpec((K, bh), lambda j: (0, j)),
            pl.BlockSpec((bh,), lambda j: (j,)),
        ],
        out_specs=pl.BlockSpec((B, bh), lambda j: (0, j)),
        out_shape=jax.ShapeDtypeStruct((B, H), jnp.float32),
    )(x, W_enc, b_enc)


def _sc_threshold_kernel(B, H):
    mesh = plsc.VectorSubcoreMesh(core_axis_name="c", subcore_axis_name="s")
    info = plsc.get_sparse_core_info()
    NC, NS, L = info.num_cores, info.num_subcores, info.num_lanes
    NW = NC * NS
    rows_per_w = B // NW
    nv_row = H // L

    NSPLIT = 8

    @functools.partial(
        pl.kernel,
        mesh=mesh,
        out_type=jax.ShapeDtypeStruct((B,), jnp.float32),
        compiler_params=pltpu.CompilerParams(needs_layout_passes=False),
        scratch_types=[
            pltpu.VMEM((H,), jnp.float32),
            pltpu.VMEM((H,), jnp.float32),
            pltpu.VMEM((NSPLIT * 256,), jnp.int32),
            pltpu.VMEM((256,), jnp.int32),
            pltpu.VMEM((rows_per_w,), jnp.float32),
            pltpu.SemaphoreType.DMA,
            pltpu.SemaphoreType.DMA,
        ],
    )
    def body(pre_hbm, t_hbm, buf0, buf1, hist1, hist2, tbuf, sem0, sem1):
        wid = lax.axis_index("s") * NC + lax.axis_index("c")
        base = wid * rows_per_w
        zeros16i = jnp.zeros((L,), jnp.int32)
        ones16i = jnp.ones((L,), jnp.int32)
        iota16 = lax.iota(jnp.int32, L)

        def hist_scan(read_vreg, nbins, S0):
            def scan(k, carry):
                S, bstar, Cab = carry
                j = (nbins // L) - 1 - k
                v = read_vreg(j)
                tot = jnp.sum(v)
                rev = lax.rev(v, dimensions=(0,))
                csum = plsc.cumsum(rev)
                crossed = (S + csum) >= TOPK
                ncross = plsc.all_reduce_population_count(crossed)[0]
                has = ncross > 0
                p = plsc.all_reduce_ffs(crossed)[0]
                bin_idx = j * L + (L - 1) - p
                cprev = jnp.sum(jnp.where(iota16 == p - 1, csum, 0))
                Cnew = S + jnp.where(p > 0, cprev, 0)
                found_now = jnp.logical_and(has, bstar < 0)
                bstar = jnp.where(found_now, bin_idx, bstar)
                Cab = jnp.where(found_now, Cnew, Cab)
                return S + tot, bstar, Cab

            _, bstar, Cab = lax.fori_loop(
                0, nbins // L, scan, (S0, jnp.int32(-1), jnp.int32(0))
            )
            return bstar, Cab

        def masked_level(buf, shift_prefix, prefix, shift_key, keymask, S0):
            def zh(u):
                hist2[pl.ds(u * L, L)] = zeros16i

            plsc.parallel_loop(0, 256 // L, step=1)(zh)

            def pbody(j):
                v = buf[pl.ds(j * L, L)]
                h = jnp.maximum(v, 0.0)
                bits = lax.bitcast_convert_type(h, jnp.int32)
                msk = lax.shift_right_logical(bits, shift_prefix) == prefix
                key = jnp.bitwise_and(
                    lax.shift_right_logical(bits, shift_key), keymask
                )
                plsc.addupdate_scatter(hist2, [key], ones16i, mask=msk)

            plsc.parallel_loop(0, nv_row, step=1)(pbody)
            nbins = keymask + 1
            return hist_scan(lambda j: hist2[pl.ds(j * L, L)], nbins, S0)

        def process_row(buf):
            def zh1(u):
                hist1[pl.ds(u * L, L)] = zeros16i

            plsc.parallel_loop(0, NSPLIT * 256 // L, step=1)(zh1)

            def p1(j):
                for u in range(NSPLIT):
                    v = buf[pl.ds((j + u) * L, L)]
                    h = jnp.maximum(v, 0.0)
                    bits = lax.bitcast_convert_type(h, jnp.int32)
                    e = lax.shift_right_logical(bits, 23)
                    plsc.addupdate_scatter(
                        hist1, [e + (u * 256)], ones16i
                    )

            plsc.parallel_loop(0, nv_row, step=NSPLIT)(p1)

            def rd1(j):
                v = hist1[pl.ds(j * L, L)]
                for u in range(1, NSPLIT):
                    v = v + hist1[pl.ds(u * 256 + j * L, L)]
                return v

            estar, C1 = hist_scan(rd1, 256, jnp.int32(0))
            b2, C2 = masked_level(buf, 23, estar, 15, 0xFF, C1)
            pref3 = lax.shift_left(estar, 8) + b2
            b3, C3 = masked_level(buf, 15, pref3, 7, 0xFF, C2)
            pref4 = lax.shift_left(pref3, 8) + b3
            b4, _ = masked_level(buf, 7, pref4, 0, 0x7F, C3)
            tbits = lax.shift_left(pref4, 7) + b4
            return lax.bitcast_convert_type(
                jnp.broadcast_to(tbits, (L,)), jnp.float32
            )

        def store_t(i, tval):
            plsc.store_scatter(
                tbuf, [jnp.broadcast_to(i, (L,))], tval, mask=iota16 == 0
            )

        pltpu.async_copy(pre_hbm.at[base], buf0, sem0)

        def pair(g, c):
            rowa = base + 2 * g
            rowb = base + 2 * g + 1
            rown = base + (2 * g + 2) % rows_per_w
            pltpu.make_async_copy(pre_hbm.at[rowa], buf0, sem0).wait()
            pltpu.async_copy(pre_hbm.at[rowb], buf1, sem1)
            store_t(2 * g, process_row(buf0))
            pltpu.make_async_copy(pre_hbm.at[rowb], buf1, sem1).wait()
            pltpu.async_copy(pre_hbm.at[rown], buf0, sem0)
            store_t(2 * g + 1, process_row(buf1))
            return c

        lax.fori_loop(0, rows_per_w // 2, pair, 0)
        pltpu.make_async_copy(pre_hbm.at[base], buf0, sem0).wait()
        pltpu.sync_copy(tbuf, t_hbm.at[pl.ds(base, rows_per_w)])

    return body


def _sc_threshold(preact):
    B, H = preact.shape
    return _sc_threshold_kernel(B, H)(preact)


def _dec_body(pre_ref, t_ref, w_ref, b_ref, hs_ref, out_ref):
    k = pl.program_id(0)
    nk = pl.num_programs(0)
    h = jnp.maximum(pre_ref[...], 0.0)
    hs = jnp.where(h >= t_ref[...], h, 0.0)
    hs_ref[...] = hs
    acc = jnp.dot(hs, w_ref[...], preferred_element_type=jnp.float32)

    @pl.when(k == 0)
    def _():
        out_ref[...] = acc

    @pl.when(k != 0)
    def _():
        out_ref[...] = out_ref[...] + acc

    @pl.when(k == nk - 1)
    def _():
        out_ref[...] = jnp.maximum(out_ref[...] + b_ref[...], 0.0)


def _decode(preact, t, W_dec, b_dec):
    B, H = preact.shape
    D = W_dec.shape[1]
    bk = 1024
    return pl.pallas_call(
        _dec_body,
        grid=(H // bk,),
        in_specs=[
            pl.BlockSpec((B, bk), lambda k: (0, k)),
            pl.BlockSpec((B, 1), lambda k: (0, 0)),
            pl.BlockSpec((bk, D), lambda k: (k, 0)),
            pl.BlockSpec((D,), lambda k: (0,)),
        ],
        out_specs=[
            pl.BlockSpec((B, bk), lambda k: (0, k)),
            pl.BlockSpec((B, D), lambda k: (0, 0)),
        ],
        out_shape=[
            jax.ShapeDtypeStruct((B, H), jnp.float32),
            jax.ShapeDtypeStruct((B, D), jnp.float32),
        ],
    )(preact, t, W_dec, b_dec)


def kernel(x, W_enc, W_dec, b_enc, b_dec):
    preact = _encode(x, W_enc, b_enc)
    t = _sc_threshold(preact)
    hidden_sparse, out = _decode(preact, t.reshape(-1, 1), W_dec, b_dec)
    return (out, hidden_sparse, preact)


# speedup vs baseline: 1.8873x; 1.4386x over previous
import functools

import jax
import jax.numpy as jnp
from jax import lax
from jax.experimental import pallas as pl
from jax.experimental.pallas import tpu as pltpu
from jax.experimental.pallas import tpu_sc as plsc

TOPK = 64


def _enc_body(x_ref, w_ref, b_ref, out_ref):
    acc = jnp.dot(x_ref[...], w_ref[...], preferred_element_type=jnp.float32)
    out_ref[...] = acc + b_ref[...]


def _encode(x, W_enc, b_enc):
    B, K = x.shape
    H = W_enc.shape[1]
    bh = 1024
    return pl.pallas_call(
        _enc_body,
        grid=(H // bh,),
        in_specs=[
            pl.BlockSpec((B, K), lambda j: (0, 0)),
            pl.BlockSpec((K, bh), lambda j: (0, j)),
            pl.BlockSpec((bh,), lambda j: (j,)),
        ],
        out_specs=pl.BlockSpec((B, bh), lambda j: (0, j)),
        out_shape=jax.ShapeDtypeStruct((B, H), jnp.float32),
    )(x, W_enc, b_enc)


def _sc_threshold_kernel(B, H):
    mesh = plsc.VectorSubcoreMesh(core_axis_name="c", subcore_axis_name="s")
    info = plsc.get_sparse_core_info()
    NC, NS, L = info.num_cores, info.num_subcores, info.num_lanes
    NW = NC * NS
    rows_per_w = B // NW
    nv_row = H // L

    NSPLIT = 8

    @functools.partial(
        pl.kernel,
        mesh=mesh,
        out_type=jax.ShapeDtypeStruct((B,), jnp.float32),
        compiler_params=pltpu.CompilerParams(needs_layout_passes=False),
        scratch_types=[
            pltpu.VMEM((H,), jnp.float32),
            pltpu.VMEM((H,), jnp.float32),
            pltpu.VMEM((NSPLIT * 256,), jnp.int32),
            pltpu.VMEM((256,), jnp.int32),
            pltpu.VMEM((rows_per_w,), jnp.float32),
            pltpu.SemaphoreType.DMA,
            pltpu.SemaphoreType.DMA,
        ],
    )
    def body(pre_hbm, t_hbm, buf0, buf1, hist1, hist2, tbuf, sem0, sem1):
        wid = lax.axis_index("s") * NC + lax.axis_index("c")
        base = wid * rows_per_w
        zeros16i = jnp.zeros((L,), jnp.int32)
        ones16i = jnp.ones((L,), jnp.int32)
        iota16 = lax.iota(jnp.int32, L)

        def in_vreg_cross(v, Sv):
            rev2 = lax.rev(v, dimensions=(0,))
            c2 = plsc.cumsum(rev2)
            crossed2 = (Sv + c2) >= TOPK
            q = plsc.all_reduce_ffs(crossed2)[0]
            off = (L - 1) - q
            cprev = jnp.sum(jnp.where(iota16 == q - 1, c2, 0))
            Cab = Sv + jnp.where(q > 0, cprev, 0)
            return off, Cab

        def hist_scan(read_vreg, nvregs, S0):
            tot = zeros16i
            for j in range(nvregs):
                tj = jnp.sum(read_vreg(j))
                tot = jnp.where(iota16 == j, tj, tot)
            srev = lax.rev(tot, dimensions=(0,))
            csum = plsc.cumsum(srev)
            crossed = (S0 + csum) >= TOPK
            p = plsc.all_reduce_ffs(crossed)[0]
            jc = (nvregs - 1) - p
            cprev = jnp.sum(jnp.where(iota16 == p - 1, csum, 0))
            Cv = S0 + jnp.where(p > 0, cprev, 0)
            off, Cab = in_vreg_cross(read_vreg(jc), Cv)
            return jc * L + off, Cab

        def masked_level(buf, shift_prefix, prefix, shift_key, keymask, S0):
            def zh(u):
                hist2[pl.ds(u * L, L)] = zeros16i

            plsc.parallel_loop(0, 256 // L, step=1)(zh)

            def pbody(j):
                for u in range(NSPLIT):
                    v = buf[pl.ds((j + u) * L, L)]
                    h = jnp.maximum(v, 0.0)
                    bits = lax.bitcast_convert_type(h, jnp.int32)
                    msk = lax.shift_right_logical(bits, shift_prefix) == prefix
                    key = jnp.bitwise_and(
                        lax.shift_right_logical(bits, shift_key), keymask
                    )
                    plsc.addupdate_scatter(hist2, [key], ones16i, mask=msk)

            plsc.parallel_loop(0, nv_row, step=NSPLIT)(pbody)
            nbins = keymask + 1
            return hist_scan(
                lambda j: hist2[pl.ds(j * L, L)], nbins // L, S0
            )

        def process_row(buf):
            def zh1(u):
                hist1[pl.ds(u * L, L)] = zeros16i

            plsc.parallel_loop(0, NSPLIT * 256 // L, step=1)(zh1)

            def p1(j):
                for u in range(NSPLIT):
                    v = buf[pl.ds((j + u) * L, L)]
                    h = jnp.maximum(v, 0.0)
                    bits = lax.bitcast_convert_type(h, jnp.int32)
                    e = lax.shift_right_logical(bits, 23)
                    plsc.addupdate_scatter(
                        hist1, [e + (u * 256)], ones16i
                    )

            plsc.parallel_loop(0, nv_row, step=NSPLIT)(p1)

            def rd1(j):
                v = hist1[pl.ds(j * L, L)]
                for u in range(1, NSPLIT):
                    v = v + hist1[pl.ds(u * 256 + j * L, L)]
                return v

            estar, C1 = hist_scan(rd1, 256 // L, jnp.int32(0))
            b2, C2 = masked_level(buf, 23, estar, 15, 0xFF, C1)
            pref3 = lax.shift_left(estar, 8) + b2
            b3, C3 = masked_level(buf, 15, pref3, 7, 0xFF, C2)
            pref4 = lax.shift_left(pref3, 8) + b3
            b4, _ = masked_level(buf, 7, pref4, 0, 0x7F, C3)
            tbits = lax.shift_left(pref4, 7) + b4
            return lax.bitcast_convert_type(
                jnp.broadcast_to(tbits, (L,)), jnp.float32
            )

        def store_t(i, tval):
            plsc.store_scatter(
                tbuf, [jnp.broadcast_to(i, (L,))], tval, mask=iota16 == 0
            )

        pltpu.async_copy(pre_hbm.at[base], buf0, sem0)

        def pair(g, c):
            rowa = base + 2 * g
            rowb = base + 2 * g + 1
            rown = base + (2 * g + 2) % rows_per_w
            pltpu.make_async_copy(pre_hbm.at[rowa], buf0, sem0).wait()
            pltpu.async_copy(pre_hbm.at[rowb], buf1, sem1)
            store_t(2 * g, process_row(buf0))
            pltpu.make_async_copy(pre_hbm.at[rowb], buf1, sem1).wait()
            pltpu.async_copy(pre_hbm.at[rown], buf0, sem0)
            store_t(2 * g + 1, process_row(buf1))
            return c

        lax.fori_loop(0, rows_per_w // 2, pair, 0)
        pltpu.make_async_copy(pre_hbm.at[base], buf0, sem0).wait()
        pltpu.sync_copy(tbuf, t_hbm.at[pl.ds(base, rows_per_w)])

    return body


def _sc_threshold(preact):
    B, H = preact.shape
    return _sc_threshold_kernel(B, H)(preact)


def _dec_body(pre_ref, t_ref, w_ref, b_ref, hs_ref, out_ref):
    k = pl.program_id(0)
    nk = pl.num_programs(0)
    h = jnp.maximum(pre_ref[...], 0.0)
    hs = jnp.where(h >= t_ref[...], h, 0.0)
    hs_ref[...] = hs
    acc = jnp.dot(hs, w_ref[...], preferred_element_type=jnp.float32)

    @pl.when(k == 0)
    def _():
        out_ref[...] = acc

    @pl.when(k != 0)
    def _():
        out_ref[...] = out_ref[...] + acc

    @pl.when(k == nk - 1)
    def _():
        out_ref[...] = jnp.maximum(out_ref[...] + b_ref[...], 0.0)


def _decode(preact, t, W_dec, b_dec):
    B, H = preact.shape
    D = W_dec.shape[1]
    bk = 1024
    return pl.pallas_call(
        _dec_body,
        grid=(H // bk,),
        in_specs=[
            pl.BlockSpec((B, bk), lambda k: (0, k)),
            pl.BlockSpec((B, 1), lambda k: (0, 0)),
            pl.BlockSpec((bk, D), lambda k: (k, 0)),
            pl.BlockSpec((D,), lambda k: (0,)),
        ],
        out_specs=[
            pl.BlockSpec((B, bk), lambda k: (0, k)),
            pl.BlockSpec((B, D), lambda k: (0, 0)),
        ],
        out_shape=[
            jax.ShapeDtypeStruct((B, H), jnp.float32),
            jax.ShapeDtypeStruct((B, D), jnp.float32),
        ],
    )(preact, t, W_dec, b_dec)


def kernel(x, W_enc, W_dec, b_enc, b_dec):
    preact = _encode(x, W_enc, b_enc)
    t = _sc_threshold(preact)
    hidden_sparse, out = _decode(preact, t.reshape(-1, 1), W_dec, b_dec)
    return (out, hidden_sparse, preact)
